# Initial kernel scaffold; baseline (speedup 1.0000x reference)
import jax
import jax.numpy as jnp
from jax.experimental import pallas as pl


def kernel(token_ids, emb_weight, proj_weight, scale):
    raise NotImplementedError("write your pallas kernel here")


# speedup vs baseline: 9.3317x; 9.3317x over previous
import functools

import jax
import jax.numpy as jnp
from jax import lax
from jax.experimental import pallas as pl
from jax.experimental.pallas import tpu as pltpu
from jax.experimental.pallas import tpu_sc as plsc

EDIM = 32
OUT_DIM = 64
PACK = 4
PEDIM = PACK * EDIM

_info = plsc.get_sparse_core_info()
_NC, _NS = _info.num_cores, _info.num_subcores
_NW = _NC * _NS


@functools.partial(jax.jit, static_argnames=("chunk",))
def _sc_gather_rows(idx4, table128, chunk=512):
    n = idx4.shape[0]
    b_per_w = n // _NW
    nchunks = b_per_w // chunk
    assert nchunks * chunk * _NW == n

    mesh = plsc.VectorSubcoreMesh(core_axis_name="c", subcore_axis_name="s")

    @functools.partial(
        pl.kernel,
        mesh=mesh,
        out_type=jax.ShapeDtypeStruct((n, PEDIM), jnp.float32),
        scratch_types=[
            pltpu.VMEM((chunk,), jnp.int32),
            pltpu.VMEM((chunk, PEDIM), jnp.float32),
            pltpu.SemaphoreType.DMA,
        ],
    )
    def gather_kernel(idx_hbm, table_hbm, out_hbm, idx_v, rows_v, sem):
        wid = lax.axis_index("s") * _NC + lax.axis_index("c")
        base = wid * b_per_w
        for i in range(nchunks):
            off = base + i * chunk
            pltpu.sync_copy(idx_hbm.at[pl.ds(off, chunk)], idx_v)
            pltpu.async_copy(table_hbm.at[idx_v], rows_v, sem).wait()
            pltpu.sync_copy(rows_v, out_hbm.at[pl.ds(off, chunk)])

    return gather_kernel(idx4, table128)


def _tc_project(g2, sel, w4, bblk=512):
    batch, hist = sel.shape
    rows = bblk * hist

    def mm_kernel(a_ref, sel_ref, w_ref, o_ref):
        a = a_ref[...]
        sel3 = sel_ref[...][:, :, None]
        acc = jnp.zeros((bblk, hist, OUT_DIM), jnp.float32)
        for k in range(PACK):
            ak = jnp.dot(a, w_ref[k], preferred_element_type=jnp.float32)
            acc = acc + jnp.where(sel3 == k, ak.reshape(bblk, hist, OUT_DIM), 0.0)
        o_ref[...] = acc

    return pl.pallas_call(
        mm_kernel,
        grid=(batch // bblk,),
        in_specs=[
            pl.BlockSpec((rows, PEDIM), lambda i: (i, 0)),
            pl.BlockSpec((bblk, hist), lambda i: (i, 0)),
            pl.BlockSpec((PACK, PEDIM, OUT_DIM), lambda i: (0, 0, 0)),
        ],
        out_specs=pl.BlockSpec((bblk, hist, OUT_DIM), lambda i: (i, 0, 0)),
        out_shape=jax.ShapeDtypeStruct((batch, hist, OUT_DIM), jnp.float32),
    )(g2, sel, w4)


def kernel(token_ids, emb_weight, proj_weight, scale):
    batch, hist = token_ids.shape
    vocab = emb_weight.shape[0]
    ids = token_ids.reshape(-1).astype(jnp.int32)
    idx4 = ids // PACK
    sel = (ids % PACK).reshape(batch, hist)
    table128 = emb_weight.reshape(vocab // PACK, PEDIM)

    base = proj_weight.T.astype(jnp.float32) * scale.astype(jnp.float32)
    w4 = jnp.zeros((PACK, PEDIM, OUT_DIM), jnp.float32)
    for k in range(PACK):
        w4 = w4.at[k, k * EDIM:(k + 1) * EDIM, :].set(base)

    g2 = _sc_gather_rows(idx4, table128)
    return _tc_project(g2, sel, w4)
